# baseline (speedup 1.0000x reference)
import functools

import jax
import jax.numpy as jnp
from jax import lax
from jax.experimental import pallas as pl
from jax.experimental.pallas import tpu as pltpu
from jax.experimental.pallas import tpu_sc as plsc

B = 16384
NZ = 1024
L = 16
NC = 2
NS = 16
NW = NC * NS

B_TC = 11264
B_SC = B - B_TC
ROWS_PER_W = B_SC // NW
R = 16
C = ROWS_PER_W // R
NJ = NZ // L

BR = 512


def _sc_body(h_hbm, idx_hbm, out_hbm, idx_v,
             in0, in1, out0, out1,
             sem_i0, sem_i1, sem_o0, sem_o1):
    wid = lax.axis_index("s") * NC + lax.axis_index("c")
    row0 = B_TC + wid * ROWS_PER_W
    pltpu.sync_copy(idx_hbm, idx_v)

    def fire_in(g, buf, sem):
        base = row0 + g * R
        pltpu.async_copy(h_hbm.at[pl.ds(base, R), :], buf, sem)

    def fire_out(g, buf, sem):
        base = g * R + wid * ROWS_PER_W
        pltpu.async_copy(buf, out_hbm.at[pl.ds(base, R), :], sem)

    def drain(buf, sem):
        pltpu.make_async_copy(h_hbm.at[pl.ds(0, R), :], buf, sem).wait()

    def compute(src, dst):
        @plsc.parallel_loop(0, NJ, step=1)
        def _jb(j):
            cidx = idx_v[pl.ds(j * L, L)]
            for r in range(R):
                rvec = jnp.full((L,), r, jnp.int32)
                dst[r, pl.ds(j * L, L)] = plsc.load_gather(
                    src, [rvec, cidx])

    fire_in(0, in0, sem_i0)

    def body(t, carry):
        g0 = 2 * t
        fire_in(g0 + 1, in1, sem_i1)
        drain(in0, sem_i0)

        @pl.when(t > 0)
        def _():
            drain(out0, sem_o0)

        compute(in0, out0)
        fire_out(g0, out0, sem_o0)

        @pl.when(g0 + 2 < C)
        def _():
            fire_in(g0 + 2, in0, sem_i0)

        drain(in1, sem_i1)

        @pl.when(t > 0)
        def _():
            drain(out1, sem_o1)

        compute(in1, out1)
        fire_out(g0 + 1, out1, sem_o1)
        return carry

    lax.fori_loop(0, C // 2, body, 0)
    drain(out0, sem_o0)
    drain(out1, sem_o1)


def _tc_body(idx_ref, h_ref, o_ref, p_ref):
    @pl.when(pl.program_id(0) == 0)
    def _():
        iota = lax.broadcasted_iota(jnp.int32, (NZ, NZ), 0)
        p_ref[...] = (iota == idx_ref[...]).astype(jnp.float32)

    o_ref[...] = jnp.dot(h_ref[...], p_ref[...],
                         preferred_element_type=jnp.float32)


def kernel(h, indices):
    idx2d = indices.reshape(1, NZ)

    mesh = plsc.VectorSubcoreMesh(core_axis_name="c", subcore_axis_name="s")
    sc = pl.kernel(
        _sc_body,
        out_type=jax.ShapeDtypeStruct((B_SC, NZ), jnp.float32),
        mesh=mesh,
        scratch_types=[
            pltpu.VMEM((NZ,), jnp.int32),
            pltpu.VMEM((R, NZ), jnp.float32),
            pltpu.VMEM((R, NZ), jnp.float32),
            pltpu.VMEM((R, NZ), jnp.float32),
            pltpu.VMEM((R, NZ), jnp.float32),
            pltpu.SemaphoreType.DMA,
            pltpu.SemaphoreType.DMA,
            pltpu.SemaphoreType.DMA,
            pltpu.SemaphoreType.DMA,
        ],
        compiler_params=pltpu.CompilerParams(needs_layout_passes=False),
    )
    out_sc = sc(h, indices)

    out_tc = pl.pallas_call(
        _tc_body,
        grid=(B_TC // BR,),
        in_specs=[
            pl.BlockSpec((1, NZ), lambda i: (0, 0)),
            pl.BlockSpec((BR, NZ), lambda i: (i, 0)),
        ],
        out_specs=pl.BlockSpec((BR, NZ), lambda i: (i, 0)),
        out_shape=jax.ShapeDtypeStruct((B_TC, NZ), jnp.float32),
        scratch_shapes=[pltpu.VMEM((NZ, NZ), jnp.float32)],
        compiler_params=pltpu.CompilerParams(
            dimension_semantics=("arbitrary",)),
    )(idx2d, h)

    return jnp.concatenate([out_tc, out_sc], axis=0)


# speedup vs baseline: 1.6254x; 1.6254x over previous
import functools

import jax
import jax.numpy as jnp
from jax import lax
from jax.experimental import pallas as pl
from jax.experimental.pallas import tpu as pltpu
from jax.experimental.pallas import tpu_sc as plsc

B = 16384
NZ = 1024
L = 16
NC = 2
NS = 16
NW = NC * NS
ROWS_PER_W = B // NW
R = 16
C = ROWS_PER_W // R
NJ = NZ // L


def _sc_body(h_hbm, idx_hbm, out_hbm, idx_v,
             in0, in1, out0, out1,
             sem_i0, sem_i1, sem_o0, sem_o1, sem_x):
    wid = lax.axis_index("s") * NC + lax.axis_index("c")
    row0 = wid * ROWS_PER_W
    pltpu.async_copy(idx_hbm, idx_v, sem_x)

    def fire_in(g, buf, sem):
        base = row0 + g * R
        pltpu.async_copy(h_hbm.at[pl.ds(base, R), :], buf, sem)

    def fire_out(g, buf, sem):
        base = row0 + g * R
        pltpu.async_copy(buf, out_hbm.at[pl.ds(base, R), :], sem)

    def drain(buf, sem):
        pltpu.make_async_copy(h_hbm.at[pl.ds(0, R), :], buf, sem).wait()

    def compute(src, dst):
        @plsc.parallel_loop(0, NJ, step=1)
        def _jb(j):
            cidx = idx_v[pl.ds(j * L, L)]
            for r in range(R):
                rvec = jnp.full((L,), r, jnp.int32)
                dst[r, pl.ds(j * L, L)] = plsc.load_gather(
                    src, [rvec, cidx])

    fire_in(0, in0, sem_i0)
    pltpu.make_async_copy(idx_hbm, idx_v, sem_x).wait()

    def body(t, carry):
        g0 = 2 * t
        fire_in(g0 + 1, in1, sem_i1)
        drain(in0, sem_i0)

        @pl.when(t > 0)
        def _():
            drain(out0, sem_o0)

        compute(in0, out0)
        fire_out(g0, out0, sem_o0)

        @pl.when(g0 + 2 < C)
        def _():
            fire_in(g0 + 2, in0, sem_i0)

        drain(in1, sem_i1)

        @pl.when(t > 0)
        def _():
            drain(out1, sem_o1)

        compute(in1, out1)
        fire_out(g0 + 1, out1, sem_o1)
        return carry

    lax.fori_loop(0, C // 2, body, 0)
    drain(out0, sem_o0)
    drain(out1, sem_o1)


def kernel(h, indices):
    mesh = plsc.VectorSubcoreMesh(core_axis_name="c", subcore_axis_name="s")
    k = pl.kernel(
        _sc_body,
        out_type=jax.ShapeDtypeStruct((B, NZ), jnp.float32),
        mesh=mesh,
        scratch_types=[
            pltpu.VMEM((NZ,), jnp.int32),
            pltpu.VMEM((R, NZ), jnp.float32),
            pltpu.VMEM((R, NZ), jnp.float32),
            pltpu.VMEM((R, NZ), jnp.float32),
            pltpu.VMEM((R, NZ), jnp.float32),
            pltpu.SemaphoreType.DMA,
            pltpu.SemaphoreType.DMA,
            pltpu.SemaphoreType.DMA,
            pltpu.SemaphoreType.DMA,
            pltpu.SemaphoreType.DMA,
        ],
        compiler_params=pltpu.CompilerParams(needs_layout_passes=False),
    )
    return k(h, indices)
